# baseline (speedup 1.0000x reference)
import functools

import jax
import jax.numpy as jnp
from jax import lax
from jax.experimental import pallas as pl
from jax.experimental.pallas import tpu as pltpu
from jax.experimental.pallas import tpu_sc as plsc

_INPUT_DIM = 1000000
_EMBED_DIM = 32
_SCALE = float(_INPUT_DIM) ** 0.5

_NC = 2
_NS = 16
_NW = _NC * _NS
_BLK = 128
_NBUF = 8


def _build_sc_gather(n_rows: int, row_len: int):
    n_blocks = n_rows // _BLK
    blocks_per_w = n_blocks // _NW
    rows_per_w = n_rows // _NW
    n_chunks = blocks_per_w * row_len
    n_rounds = n_chunks // _NBUF
    assert n_chunks % _NBUF == 0
    mesh = plsc.VectorSubcoreMesh(core_axis_name="c", subcore_axis_name="s")

    @functools.partial(
        pl.kernel,
        mesh=mesh,
        out_type=jax.ShapeDtypeStruct(
            (row_len, _EMBED_DIM // 8, n_blocks, 8, _BLK), jnp.float32
        ),
        scratch_types=[
            pltpu.VMEM((rows_per_w, row_len), jnp.int32),
            [pltpu.VMEM((_BLK,), jnp.int32)] * _NBUF,
            [pltpu.VMEM((_BLK, _EMBED_DIM), jnp.float32)] * _NBUF,
            [pltpu.VMEM((_EMBED_DIM // 8, 8, _BLK), jnp.float32)] * _NBUF,
            [pltpu.SemaphoreType.DMA] * _NBUF,
            [pltpu.SemaphoreType.DMA] * _NBUF,
        ],
        compiler_params=pltpu.CompilerParams(
            use_tc_tiling_on_sc=False, needs_layout_passes=False
        ),
    )
    def k(table_hbm, idx_hbm, out_hbm, idx_v, cidx, grows, orows, gsem, osem):
        wid = lax.axis_index("s") * _NC + lax.axis_index("c")
        base_row = wid * rows_per_w
        base_blk = wid * blocks_per_w
        pltpu.sync_copy(idx_hbm.at[pl.ds(base_row, rows_per_w)], idx_v)

        iota16 = lax.iota(jnp.int32, 16)
        ds_v = iota16 & 7
        g_lo = lax.shift_right_logical(iota16, 3)
        g_hi = g_lo + 2

        def chunk_sl(c):
            return c // row_len, c % row_len

        def build_cidx(b, c):
            lj, s = chunk_sl(c)
            s_v = jnp.full((16,), s, jnp.int32)

            @pl.loop(0, _BLK, step=16)
            def _(il):
                rows = lj * _BLK + il + iota16
                cidx[b][pl.ds(il, 16)] = plsc.load_gather(idx_v, [rows, s_v])

        def start_gather(b):
            pltpu.async_copy(table_hbm.at[cidx[b]], grows[b], gsem[b])

        def wait_gather(b):
            pltpu.make_async_copy(
                table_hbm.at[pl.ds(0, _BLK)], grows[b], gsem[b]
            ).wait()

        def start_out(b, c):
            lj, s = chunk_sl(c)
            pltpu.async_copy(
                orows[b], out_hbm.at[s, :, base_blk + lj], osem[b]
            )

        def wait_out(b):
            pltpu.make_async_copy(orows[b], out_hbm.at[0, :, 0], osem[b]).wait()

        def transpose_scale(b):
            for g in range(_EMBED_DIM // 8):

                @pl.loop(0, _BLK, step=16)
                def _(il):
                    rows = il + iota16
                    for ds in range(8):
                        d_v = jnp.full((16,), g * 8 + ds, jnp.int32)
                        v = plsc.load_gather(grows[b], [rows, d_v]) * _SCALE
                        orows[b][g, ds, pl.ds(il, 16)] = v

        for b in range(_NBUF):
            build_cidx(b, b)
            start_gather(b)

        @pl.loop(0, n_rounds)
        def _(g):
            for b in range(_NBUF):
                c = g * _NBUF + b
                wait_gather(b)

                @pl.when(g > 0)
                def _():
                    wait_out(b)

                transpose_scale(b)

                @pl.when(c + _NBUF < n_chunks)
                def _():
                    build_cidx(b, c + _NBUF)
                    start_gather(b)

                start_out(b, c)

        for b in range(_NBUF):
            wait_out(b)

    return k


def kernel(x, table):
    b, s = x.shape
    out5 = _build_sc_gather(b, s)(table, x.astype(jnp.int32))
    return out5.transpose(2, 4, 0, 1, 3).reshape(b, s, _EMBED_DIM)


# speedup vs baseline: 1.5506x; 1.5506x over previous
import functools

import jax
import jax.numpy as jnp
from jax import lax
from jax.experimental import pallas as pl
from jax.experimental.pallas import tpu as pltpu
from jax.experimental.pallas import tpu_sc as plsc

_INPUT_DIM = 1000000
_EMBED_DIM = 32
_SCALE = float(_INPUT_DIM) ** 0.5

_NC = 2
_NS = 16
_NW = _NC * _NS
_BLK = 128
_NBUF = 8


def _build_sc_gather(n_rows: int, row_len: int):
    n_blocks = n_rows // _BLK
    blocks_per_w = n_blocks // _NW
    rows_per_w = n_rows // _NW
    n_chunks = blocks_per_w * row_len
    n_rounds = n_chunks // _NBUF
    assert n_chunks % _NBUF == 0
    mesh = plsc.VectorSubcoreMesh(core_axis_name="c", subcore_axis_name="s")

    @functools.partial(
        pl.kernel,
        mesh=mesh,
        out_type=jax.ShapeDtypeStruct(
            (row_len, _EMBED_DIM // 8, n_blocks, 8, _BLK), jnp.float32
        ),
        scratch_types=[
            pltpu.VMEM((rows_per_w, row_len), jnp.int32),
            [pltpu.VMEM((_BLK,), jnp.int32)] * _NBUF,
            [pltpu.VMEM((_BLK, _EMBED_DIM), jnp.float32)] * _NBUF,
            [pltpu.VMEM((_EMBED_DIM // 8, 8, _BLK + 1), jnp.float32)] * _NBUF,
            [pltpu.SemaphoreType.DMA] * _NBUF,
            [pltpu.SemaphoreType.DMA] * _NBUF,
        ],
        compiler_params=pltpu.CompilerParams(
            use_tc_tiling_on_sc=False, needs_layout_passes=False
        ),
    )
    def k(table_hbm, idx_hbm, out_hbm, idx_v, cidx, grows, orows, gsem, osem):
        wid = lax.axis_index("s") * _NC + lax.axis_index("c")
        base_row = wid * rows_per_w
        base_blk = wid * blocks_per_w
        pltpu.sync_copy(idx_hbm.at[pl.ds(base_row, rows_per_w)], idx_v)

        iota16 = lax.iota(jnp.int32, 16)
        ds_v = iota16 & 7
        g_lo = lax.shift_right_logical(iota16, 3)
        g_hi = g_lo + 2

        def chunk_sl(c):
            return c // row_len, c % row_len

        def build_cidx(b, c):
            lj, s = chunk_sl(c)
            s_v = jnp.full((16,), s, jnp.int32)

            @pl.loop(0, _BLK, step=16)
            def _(il):
                rows = lj * _BLK + il + iota16
                cidx[b][pl.ds(il, 16)] = plsc.load_gather(idx_v, [rows, s_v])

        def start_gather(b):
            pltpu.async_copy(table_hbm.at[cidx[b]], grows[b], gsem[b])

        def wait_gather(b):
            pltpu.make_async_copy(
                table_hbm.at[pl.ds(0, _BLK)], grows[b], gsem[b]
            ).wait()

        def start_out(b, c):
            lj, s = chunk_sl(c)
            pltpu.async_copy(
                orows[b].at[:, :, pl.ds(0, _BLK)],
                out_hbm.at[s, :, base_blk + lj],
                osem[b],
            )

        def wait_out(b):
            pltpu.make_async_copy(
                orows[b].at[:, :, pl.ds(0, _BLK)], out_hbm.at[0, :, 0], osem[b]
            ).wait()

        def transpose_scale(b):
            @pl.loop(0, _BLK, step=4)
            def _(il):
                for r in range(4):
                    il_v = jnp.full((16,), il + r, jnp.int32)
                    v_lo = grows[b][il + r, pl.ds(0, 16)] * _SCALE
                    plsc.store_scatter(orows[b], [g_lo, ds_v, il_v], v_lo)
                    v_hi = grows[b][il + r, pl.ds(16, 16)] * _SCALE
                    plsc.store_scatter(orows[b], [g_hi, ds_v, il_v], v_hi)

        for b in range(_NBUF):
            build_cidx(b, b)
            start_gather(b)

        for b in range(_NBUF):
            wait_gather(b)
            transpose_scale(b)
            build_cidx(b, _NBUF + b)
            start_gather(b)
            start_out(b, b)

        @pl.loop(1, n_rounds - 1)
        def _(g):
            for b in range(_NBUF):
                c = g * _NBUF + b
                wait_gather(b)
                wait_out(b)
                transpose_scale(b)
                build_cidx(b, c + _NBUF)
                start_gather(b)
                start_out(b, c)

        for b in range(_NBUF):
            c = (n_rounds - 1) * _NBUF + b
            wait_gather(b)
            wait_out(b)
            transpose_scale(b)
            start_out(b, c)

        for b in range(_NBUF):
            wait_out(b)

    return k


def kernel(x, table):
    b, s = x.shape
    out5 = _build_sc_gather(b, s)(table, x.astype(jnp.int32))
    return out5.transpose(2, 4, 0, 1, 3).reshape(b, s, _EMBED_DIM)
